# baseline (speedup 1.0000x reference)
import functools

import jax
import jax.numpy as jnp
from jax import lax
from jax.experimental import pallas as pl
from jax.experimental.pallas import tpu as pltpu
from jax.experimental.pallas import tpu_sc as plsc

N_USERS = 1000000
N_ITEMS = 100000
D = 64
B = 16384

NC = 2
NS = 16
NW = NC * NS
B_PER_W = B // NW
QB = 128
N_Q = B_PER_W // QB


def _mf_kernel(u_hbm, i_hbm, ut_hbm, it_hbm,
               pred_hbm, p_hbm, q_hbm,
               idx_u, idx_i, p_big, q_big, p_pair, q_pair,
               pred_v, sem_u, sem_i):
    wid = lax.axis_index("s") * NC + lax.axis_index("c")
    base = wid * B_PER_W
    lanes = lax.iota(jnp.int32, 16)

    pltpu.sync_copy(u_hbm.at[pl.ds(base, B_PER_W)], idx_u)
    pltpu.sync_copy(i_hbm.at[pl.ds(base, B_PER_W)], idx_i)

    for t in range(N_Q):
        cu = pltpu.async_copy(
            ut_hbm.at[idx_u.at[pl.ds(t * QB, QB)]], p_big, sem_u)
        ci = pltpu.async_copy(
            it_hbm.at[idx_i.at[pl.ds(t * QB, QB)]], q_big, sem_i)
        cu.wait()
        ci.wait()

        def body(g, carry):
            out = jnp.zeros((16,), jnp.float32)
            for r in range(16):
                b = g * 16 + r
                pr = g * 8 + r // 2
                po = (r % 2) * D
                acc = None
                for c in range(D // 16):
                    pv = p_big[b, pl.ds(c * 16, 16)]
                    qv = q_big[b, pl.ds(c * 16, 16)]
                    p_pair[pr, pl.ds(po + c * 16, 16)] = pv
                    q_pair[pr, pl.ds(po + c * 16, 16)] = qv
                    acc = pv * qv if acc is None else acc + pv * qv
                out = jnp.where(lanes == r, jnp.sum(acc), out)
            pred_v[pl.ds(t * QB + g * 16, 16)] = out
            return carry

        lax.fori_loop(0, QB // 16, body, 0)

        pair_base = pl.multiple_of((base + t * QB) // 2, 64)
        pltpu.sync_copy(p_pair, p_hbm.at[pl.ds(pair_base, QB // 2)])
        pltpu.sync_copy(q_pair, q_hbm.at[pl.ds(pair_base, QB // 2)])

    pltpu.sync_copy(pred_v, pred_hbm.at[pl.ds(base, B_PER_W)])


@jax.jit
def _mf(u, i, user_table, item_table):
    mesh = plsc.VectorSubcoreMesh(core_axis_name="c", subcore_axis_name="s")
    run = functools.partial(
        pl.kernel,
        out_type=(
            jax.ShapeDtypeStruct((B,), jnp.float32),
            jax.ShapeDtypeStruct((B // 2, 2 * D), jnp.float32),
            jax.ShapeDtypeStruct((B // 2, 2 * D), jnp.float32),
        ),
        mesh=mesh,
        compiler_params=pltpu.CompilerParams(needs_layout_passes=False),
        scratch_types=[
            pltpu.VMEM((B_PER_W,), jnp.int32),
            pltpu.VMEM((B_PER_W,), jnp.int32),
            pltpu.VMEM((QB, 2 * D), jnp.float32),
            pltpu.VMEM((QB, 2 * D), jnp.float32),
            pltpu.VMEM((QB // 2, 2 * D), jnp.float32),
            pltpu.VMEM((QB // 2, 2 * D), jnp.float32),
            pltpu.VMEM((B_PER_W,), jnp.float32),
            pltpu.SemaphoreType.DMA,
            pltpu.SemaphoreType.DMA,
        ],
    )(_mf_kernel)
    ut_w = jnp.concatenate([user_table, user_table[:, :D]], axis=1)
    it_w = jnp.concatenate([item_table, item_table[:, :D]], axis=1)
    pred, p, q = run(u, i, ut_w, it_w)
    return pred, p.reshape(B, 1, D), q.reshape(B, D, 1)


def kernel(u, i, user_table, item_table):
    return _mf(u, i, user_table, item_table)


# speedup vs baseline: 1.2581x; 1.2581x over previous
import functools

import jax
import jax.numpy as jnp
from jax import lax
from jax.experimental import pallas as pl
from jax.experimental.pallas import tpu as pltpu
from jax.experimental.pallas import tpu_sc as plsc

N_USERS = 1000000
N_ITEMS = 100000
D = 64
B = 16384

NC = 2
NS = 16
NW = NC * NS
B_PER_W = B // NW
QB = 128
N_Q = B_PER_W // QB


def _mf_kernel(u_hbm, i_hbm, ut_hbm, it_hbm,
               pred_hbm, p_hbm, q_hbm,
               idx_u, idx_i, p_big, q_big, p_pair, q_pair,
               pred_v, sem_u, sem_i):
    wid = lax.axis_index("s") * NC + lax.axis_index("c")
    base = wid * B_PER_W
    lanes = lax.iota(jnp.int32, 16)

    pltpu.sync_copy(u_hbm.at[pl.ds(base, B_PER_W)], idx_u)
    pltpu.sync_copy(i_hbm.at[pl.ds(base, B_PER_W)], idx_i)

    for t in range(N_Q):
        cu = pltpu.async_copy(
            ut_hbm.at[idx_u.at[pl.ds(t * QB, QB)]], p_big, sem_u)
        ci = pltpu.async_copy(
            it_hbm.at[idx_i.at[pl.ds(t * QB, QB)]], q_big, sem_i)
        cu.wait()
        ci.wait()

        def body(g, carry):
            out = jnp.zeros((16,), jnp.float32)
            for r in range(16):
                b = g * 16 + r
                pr = g * 8 + r // 2
                po = (r % 2) * D
                acc = None
                for c in range(D // 16):
                    pv = p_big[b, pl.ds(c * 16, 16)]
                    qv = q_big[b, pl.ds(c * 16, 16)]
                    p_pair[pr, pl.ds(po + c * 16, 16)] = pv
                    q_pair[pr, pl.ds(po + c * 16, 16)] = qv
                    acc = pv * qv if acc is None else acc + pv * qv
                out = jnp.where(lanes == r, jnp.sum(acc), out)
            pred_v[pl.ds(t * QB + g * 16, 16)] = out
            return carry

        lax.fori_loop(0, QB // 16, body, 0)

        pair_base = pl.multiple_of((base + t * QB) // 2, 64)
        pltpu.sync_copy(p_pair, p_hbm.at[pl.ds(pair_base, QB // 2)])
        pltpu.sync_copy(q_pair, q_hbm.at[pl.ds(pair_base, QB // 2)])

    pltpu.sync_copy(pred_v, pred_hbm.at[pl.ds(base, B_PER_W)])


@jax.jit
def _mf(u, i, user_table, item_table):
    mesh = plsc.VectorSubcoreMesh(core_axis_name="c", subcore_axis_name="s")
    run = functools.partial(
        pl.kernel,
        out_type=(
            jax.ShapeDtypeStruct((B,), jnp.float32),
            jax.ShapeDtypeStruct((B // 2, 2 * D), jnp.float32),
            jax.ShapeDtypeStruct((B // 2, 2 * D), jnp.float32),
        ),
        mesh=mesh,
        compiler_params=pltpu.CompilerParams(needs_layout_passes=False),
        scratch_types=[
            pltpu.VMEM((B_PER_W,), jnp.int32),
            pltpu.VMEM((B_PER_W,), jnp.int32),
            pltpu.VMEM((QB, 2 * D), jnp.float32),
            pltpu.VMEM((QB, 2 * D), jnp.float32),
            pltpu.VMEM((QB // 2, 2 * D), jnp.float32),
            pltpu.VMEM((QB // 2, 2 * D), jnp.float32),
            pltpu.VMEM((B_PER_W,), jnp.float32),
            pltpu.SemaphoreType.DMA,
            pltpu.SemaphoreType.DMA,
        ],
    )(_mf_kernel)
    ut_w = jnp.pad(user_table, ((0, 0), (0, 2 * D - D)))
    it_w = jnp.pad(item_table, ((0, 0), (0, 2 * D - D)))
    pred, p, q = run(u, i, ut_w, it_w)
    return pred, p.reshape(B, 1, D), q.reshape(B, D, 1)


def kernel(u, i, user_table, item_table):
    return _mf(u, i, user_table, item_table)
